# baseline (speedup 1.0000x reference)
import jax
import jax.numpy as jnp
from jax.experimental import pallas as pl
from jax.experimental.pallas import tpu as pltpu

_N, _P, _S = 16, 4096, 64


def _tent_body(d_ref, s_ref, o_ref):
    sam = s_ref[...].reshape(_S, 1)
    for k in range(8):
        d = d_ref[k]
        x = d[0:1, :]
        y = d[1:2, :]
        m = 0.5 * (x + y)
        h = 0.5 * (y - x)
        o_ref[k] = jnp.maximum(h - jnp.abs(sam - m), 0.0)


def kernel(diagrams, samples):
    dt = jnp.transpose(diagrams, (0, 2, 1))
    out_t = pl.pallas_call(
        _tent_body,
        grid=(_N // 8,),
        in_specs=[
            pl.BlockSpec((8, 2, _P), lambda i: (i, 0, 0)),
            pl.BlockSpec((_S,), lambda i: (0,)),
        ],
        out_specs=pl.BlockSpec((8, _S, _P), lambda i: (i, 0, 0)),
        out_shape=jax.ShapeDtypeStruct((_N, _S, _P), jnp.float32),
    )(dt, samples)
    return jnp.transpose(out_t, (0, 2, 1))
